# Initial kernel scaffold; baseline (speedup 1.0000x reference)
import jax
import jax.numpy as jnp
from jax.experimental import pallas as pl


def kernel(output, target):
    raise NotImplementedError("write your pallas kernel here")


# speedup vs baseline: 32.9666x; 32.9666x over previous
import functools

import jax
import jax.numpy as jnp
from jax import lax
from jax.experimental import pallas as pl
from jax.experimental.pallas import tpu as pltpu

NUM_CLASSES = 255
LOWER = -20.0
UPPER = 20.0
H = (UPPER - LOWER) / (NUM_CLASSES - 1)

ROWS = 1024 * 64
BLOCK_R = 2048


def _body(x_ref, t_ref, acc_ref):
    i = pl.program_id(0)
    x = x_ref[...]
    t = t_ref[...]
    tl = jnp.sign(t) * jnp.log1p(jnp.abs(t))

    m = jnp.max(x, axis=-1, keepdims=True)
    s = jnp.sum(jnp.exp(x - m), axis=-1, keepdims=True)
    lse = m + jnp.log(s)

    j = lax.broadcasted_iota(jnp.int32, (1, NUM_CLASSES), 1).astype(jnp.float32)
    binj = LOWER + j * H
    in_range = (tl > LOWER).astype(jnp.float32)
    tent = jnp.clip(1.0 - jnp.abs(tl - binj) * (1.0 / H), 0.0, 1.0) * in_range
    dot = jnp.sum(x * tent, axis=-1, keepdims=True)

    psum = in_range * (1.0 - jnp.clip((tl - UPPER) * (1.0 / H), 0.0, 1.0))

    part = jnp.sum(psum * lse - dot, keepdims=True)

    @pl.when(i == 0)
    def _():
        acc_ref[...] = jnp.zeros((1, 1), jnp.float32)

    acc_ref[...] += part


@jax.jit
def kernel(output, target):
    x = output.reshape(ROWS, NUM_CLASSES)
    t = target.reshape(ROWS, 1)
    grid = (ROWS // BLOCK_R,)
    acc = pl.pallas_call(
        _body,
        grid=grid,
        in_specs=[
            pl.BlockSpec((BLOCK_R, NUM_CLASSES), lambda i: (i, 0)),
            pl.BlockSpec((BLOCK_R, 1), lambda i: (i, 0)),
        ],
        out_specs=pl.BlockSpec((1, 1), lambda i: (0, 0)),
        out_shape=jax.ShapeDtypeStruct((1, 1), jnp.float32),
        compiler_params=pltpu.CompilerParams(
            dimension_semantics=("arbitrary",),
        ),
    )(x, t)
    return acc[0, 0] / ROWS
